# Initial kernel scaffold; baseline (speedup 1.0000x reference)
import jax
import jax.numpy as jnp
from jax.experimental import pallas as pl


def kernel(x, edge_index, edge_weight, Wq, bq, Wk, bk, Wv, bv, Wd, bd, Wo, bo):
    raise NotImplementedError("write your pallas kernel here")


# speedup vs baseline: 4.1544x; 4.1544x over previous
import functools
import math

import jax
import jax.numpy as jnp
from jax import lax
from jax.experimental import pallas as pl
from jax.experimental.pallas import tpu as pltpu
from jax.experimental.pallas import tpu_sc as plsc

_N = 10000
_NP = 10240
_E = 320000
_D = 128
_H = 8
_HD = 16
_NC = 2
_NS = 16
_NW = _NC * _NS
_EPT = _E // _NW
_B = 80
_NCH = _EPT // _B
_NG = _B // 16
_RPT = _N // _NS
_RC = 125

_F32 = jnp.float32
_I32 = jnp.int32


def _dense_body(x_ref, wq_ref, wk_ref, wv_ref, bq_ref, bk_ref, bv_ref,
                q_ref, k_ref, v_ref, xm_ref):
    xb = x_ref[...]
    q_ref[...] = jnp.dot(xb, wq_ref[...], preferred_element_type=_F32) + bq_ref[...]
    k_ref[...] = jnp.dot(xb, wk_ref[...], preferred_element_type=_F32) + bk_ref[...]
    v_ref[...] = jnp.dot(xb, wv_ref[...], preferred_element_type=_F32) + bv_ref[...]
    xm_ref[...] = jnp.mean(xb, axis=1).reshape(8, 128)


_BN = 1024


@jax.jit
def _dense(xp, wqt, wkt, wvt, bq2, bk2, bv2):
    grid = _NP // _BN
    return pl.pallas_call(
        _dense_body,
        grid=(grid,),
        in_specs=[
            pl.BlockSpec((_BN, _D), lambda i: (i, 0)),
            pl.BlockSpec((_D, _D), lambda i: (0, 0)),
            pl.BlockSpec((_D, _D), lambda i: (0, 0)),
            pl.BlockSpec((_D, _D), lambda i: (0, 0)),
            pl.BlockSpec((1, _D), lambda i: (0, 0)),
            pl.BlockSpec((1, _D), lambda i: (0, 0)),
            pl.BlockSpec((1, _D), lambda i: (0, 0)),
        ],
        out_specs=[
            pl.BlockSpec((_BN, _D), lambda i: (i, 0)),
            pl.BlockSpec((_BN, _D), lambda i: (i, 0)),
            pl.BlockSpec((_BN, _D), lambda i: (i, 0)),
            pl.BlockSpec((_BN // _D, _D), lambda i: (i, 0)),
        ],
        out_shape=[
            jax.ShapeDtypeStruct((_NP, _D), _F32),
            jax.ShapeDtypeStruct((_NP, _D), _F32),
            jax.ShapeDtypeStruct((_NP, _D), _F32),
            jax.ShapeDtypeStruct((_NP // _D, _D), _F32),
        ],
    )(xp, wqt, wkt, wvt, bq2, bk2, bv2)


def _final_body(p0_ref, p1_ref, wot_ref, bo_ref, o_ref):
    acc = p0_ref[...] + p1_ref[...]
    o_ref[...] = jnp.dot(acc, wot_ref[...], preferred_element_type=_F32) + bo_ref[...]


_FBN = 400


@jax.jit
def _final(p0, p1, wot, bo2):
    grid = _N // _FBN
    return pl.pallas_call(
        _final_body,
        grid=(grid,),
        in_specs=[
            pl.BlockSpec((_FBN, _D), lambda i: (i, 0)),
            pl.BlockSpec((_FBN, _D), lambda i: (i, 0)),
            pl.BlockSpec((_D, _D), lambda i: (0, 0)),
            pl.BlockSpec((1, _D), lambda i: (0, 0)),
        ],
        out_specs=pl.BlockSpec((_FBN, _D), lambda i: (i, 0)),
        out_shape=jax.ShapeDtypeStruct((_N, _D), _F32),
    )(p0, p1, wot, bo2)


def _sc_mesh():
    return plsc.VectorSubcoreMesh(
        core_axis_name="c", subcore_axis_name="s",
        num_cores=_NC, num_subcores=_NS)


def _phase_a_fn():
    out_type = (
        jax.ShapeDtypeStruct((_NW, _NCH, _H, _B), _F32),
        jax.ShapeDtypeStruct((_NW, 2, 16), _F32),
    )
    scratch = [
        pltpu.VMEM((_B,), _I32),
        pltpu.VMEM((_B,), _I32),
        pltpu.VMEM((_B,), _F32),
        pltpu.VMEM((_B, _D), _F32),
        pltpu.VMEM((_B, _D), _F32),
        pltpu.VMEM((_NP // _D, _D), _F32),
        pltpu.VMEM((2, 16), _F32),
        pltpu.VMEM((_H, _B), _F32),
        pltpu.VMEM((_H, 16), _F32),
        pltpu.VMEM((_H, 16), _F32),
        pltpu.VMEM((2, 16), _F32),
        pltpu.SemaphoreType.DMA,
        pltpu.SemaphoreType.DMA,
    ]

    @functools.partial(pl.kernel, out_type=out_type, mesh=_sc_mesh(),
                       scratch_types=scratch,
                       compiler_params=pltpu.CompilerParams(
                           needs_layout_passes=False, use_tc_tiling_on_sc=False))
    def phase_a(q_hbm, k_hbm, xm_hbm, row_hbm, col_hbm, ew_hbm, aux_hbm,
                scores_hbm, mz_hbm,
                row_v, col_v, ew_v, qbuf, kbuf, xm_v, aux_v, sc_v,
                m_buf, z_buf, mz_out, sem0, sem1):
        cid = lax.axis_index("c")
        sid = lax.axis_index("s")
        wid = cid * _NS + sid
        base = wid * _EPT

        pltpu.sync_copy(xm_hbm, xm_v)
        pltpu.sync_copy(aux_hbm, aux_v)

        neg = jnp.full((16,), -1e30, _F32)
        zero = jnp.zeros((16,), _F32)
        for h in range(_H):
            m_buf[h, :] = neg
            z_buf[h, :] = zero
        iota = lax.iota(_I32, 16)
        aux0 = aux_v[0, :]
        aux1 = aux_v[1, :]
        wds = [jnp.full((16,), aux0[h], _F32) for h in range(_H)]
        bds = [jnp.full((16,), aux1[h], _F32) for h in range(_H)]

        def chunk_body(c, _):
            off = base + c * _B
            pltpu.sync_copy(row_hbm.at[pl.ds(off, _B)], row_v)
            pltpu.sync_copy(col_hbm.at[pl.ds(off, _B)], col_v)
            pltpu.sync_copy(ew_hbm.at[pl.ds(off, _B)], ew_v)
            cq = pltpu.async_copy(q_hbm.at[row_v], qbuf, sem0)
            ck = pltpu.async_copy(k_hbm.at[col_v], kbuf, sem1)
            cq.wait()
            ck.wait()

            def group_body(g, _):
                gs = g * 16
                row16 = row_v[pl.ds(gs, 16)]
                col16 = col_v[pl.ds(gs, 16)]
                ew16 = ew_v[pl.ds(gs, 16)]
                dsc = (plsc.load_gather(
                           xm_v, [lax.shift_right_logical(col16, 7),
                                  lax.bitwise_and(col16, 127)])
                       - plsc.load_gather(
                           xm_v, [lax.shift_right_logical(row16, 7),
                                  lax.bitwise_and(row16, 127)]))
                lidx = iota + gs
                for h in range(_H):
                    acc = dsc * wds[h] + ew16 + bds[h]
                    for d in range(_HD):
                        cix = jnp.full((16,), h * _HD + d, _I32)
                        acc = acc + (plsc.load_gather(qbuf, [lidx, cix])
                                     * plsc.load_gather(kbuf, [lidx, cix]))
                    sc_v[h, pl.ds(gs, 16)] = acc
                    m_old = m_buf[h, :]
                    m_new = jnp.maximum(m_old, acc)
                    z_buf[h, :] = (z_buf[h, :] * jnp.exp(m_old - m_new)
                                   + jnp.exp(acc - m_new))
                    m_buf[h, :] = m_new
                return 0

            lax.fori_loop(0, _NG, group_body, 0)
            pltpu.sync_copy(sc_v, scores_hbm.at[wid, c])
            return 0

        lax.fori_loop(0, _NCH, chunk_body, 0)

        mvec = jnp.full((16,), -1e30, _F32)
        zvec = jnp.zeros((16,), _F32)
        for h in range(_H):
            mv = m_buf[h, :]
            zv = z_buf[h, :]
            mt = jnp.max(mv)
            zt = jnp.sum(zv * jnp.exp(mv - jnp.full((16,), mt, _F32)))
            sel = iota == h
            mvec = jnp.where(sel, jnp.full((16,), mt, _F32), mvec)
            zvec = jnp.where(sel, jnp.full((16,), zt, _F32), zvec)
        mz_out[0, :] = mvec
        mz_out[1, :] = zvec
        pltpu.sync_copy(mz_out, mz_hbm.at[wid])

    return phase_a


def _phase_b_fn():
    out_type = jax.ShapeDtypeStruct((_NC, _N, _D), _F32)
    scratch = [
        pltpu.VMEM((_B,), _I32),
        pltpu.VMEM((_B,), _I32),
        pltpu.VMEM((_H, _B), _F32),
        pltpu.VMEM((_B, _D), _F32),
        pltpu.VMEM((_B, _D), _F32),
        pltpu.VMEM((_NW, 2, 16), _F32),
        pltpu.VMEM((_RC, _D), _F32),
        pltpu.VMEM_SHARED((_N, _D), _F32),
        pltpu.SemaphoreType.DMA,
    ]

    @functools.partial(pl.kernel, out_type=out_type, mesh=_sc_mesh(),
                       scratch_types=scratch,
                       compiler_params=pltpu.CompilerParams(
                           needs_layout_passes=False, use_tc_tiling_on_sc=False))
    def phase_b(v_hbm, scores_hbm, mz_hbm, row_hbm, col_hbm,
                outp_hbm,
                row_v, col_v, sc_v, vbuf, msg, mz_all,
                zrow_v, accum, sem0):
        cid = lax.axis_index("c")
        sid = lax.axis_index("s")
        wid = cid * _NS + sid
        base = wid * _EPT

        pltpu.sync_copy(mz_hbm, mz_all)
        gm = mz_all[0, 0, :]
        for t in range(1, _NW):
            gm = jnp.maximum(gm, mz_all[t, 0, :])
        gz = jnp.zeros((16,), _F32)
        for t in range(_NW):
            gz = gz + mz_all[t, 1, :] * jnp.exp(mz_all[t, 0, :] - gm)
        ginv = 1.0 / gz
        gms = [jnp.full((16,), gm[h], _F32) for h in range(_H)]
        gis = [jnp.full((16,), ginv[h], _F32) for h in range(_H)]

        z16 = jnp.zeros((16,), _F32)

        def zr_body(r, _):
            for j in range(_D // 16):
                zrow_v[r, pl.ds(j * 16, 16)] = z16
            return 0

        lax.fori_loop(0, _RC, zr_body, 0)
        for j in range(_RPT // _RC):
            pltpu.sync_copy(zrow_v,
                            accum.at[pl.ds(sid * _RPT + j * _RC, _RC)])
        plsc.subcore_barrier()

        def chunk_body(c, _):
            off = base + c * _B
            pltpu.sync_copy(row_hbm.at[pl.ds(off, _B)], row_v)
            pltpu.sync_copy(col_hbm.at[pl.ds(off, _B)], col_v)
            pltpu.sync_copy(scores_hbm.at[wid, c], sc_v)
            pltpu.async_copy(v_hbm.at[col_v], vbuf, sem0).wait()

            def group_body(g, _):
                gs = g * 16
                attns = []
                for h in range(_H):
                    s = sc_v[h, pl.ds(gs, 16)]
                    attns.append(jnp.exp(s - gms[h]) * gis[h])
                for b in range(16):
                    e = gs + b
                    for h in range(_H):
                        asc = jnp.full((16,), attns[h][b], _F32)
                        msg[e, pl.ds(h * _HD, _HD)] = (
                            vbuf[e, pl.ds(h * _HD, _HD)] * asc)
                return 0

            lax.fori_loop(0, _NG, group_body, 0)
            pltpu.sync_copy(msg, accum.at[row_v], add=True)
            return 0

        lax.fori_loop(0, _NCH, chunk_body, 0)
        plsc.subcore_barrier()

        for j in range(_RPT // _RC):
            rb = sid * _RPT + j * _RC
            pltpu.sync_copy(accum.at[pl.ds(rb, _RC)],
                            outp_hbm.at[cid, pl.ds(rb, _RC)])

    return phase_b


_PHASE_A = _phase_a_fn()
_PHASE_B = _phase_b_fn()


@jax.jit
def _run(x, edge_index, edge_weight, Wq, bq, Wk, bk, Wv, bv, Wd, bd, Wo, bo):
    scale = 1.0 / math.sqrt(_HD)
    xp = jnp.pad(x, ((0, _NP - _N), (0, 0)))
    qs, ks, vs, xm = _dense(
        xp,
        Wq.T * scale, Wk.T, Wv.T,
        (bq * scale).reshape(1, _D), bk.reshape(1, _D), bv.reshape(1, _D),
    )
    row = edge_index[0]
    col = edge_index[1]
    aux = (jnp.zeros((2, 16), _F32)
           .at[0, :_H].set(Wd[:, 0])
           .at[1, :_H].set(bd))
    scores, mz = _PHASE_A(qs, ks, xm, row, col, edge_weight, aux)
    partials = _PHASE_B(vs, scores, mz, row, col)
    return _final(partials[0], partials[1], Wo.T, bo.reshape(1, _D))


def kernel(x, edge_index, edge_weight, Wq, bq, Wk, bk, Wv, bv, Wd, bd, Wo, bo):
    return _run(x, edge_index, edge_weight, Wq, bq, Wk, bk, Wv, bv,
                Wd, bd, Wo, bo)


# speedup vs baseline: 5.5932x; 1.3463x over previous
import functools
import math

import jax
import jax.numpy as jnp
from jax import lax
from jax.experimental import pallas as pl
from jax.experimental.pallas import tpu as pltpu
from jax.experimental.pallas import tpu_sc as plsc

_N = 10000
_NP = 10240
_E = 320000
_D = 128
_H = 8
_HD = 16
_NC = 2
_NS = 16
_NW = _NC * _NS
_EPT = _E // _NW
_B = 80
_NCH = _EPT // _B
_NG = _B // 16
_RPT = _N // _NS
_RC = 125
_HF = _D // 2
_HH = _H // 2

_F32 = jnp.float32
_I32 = jnp.int32


def _dense_body(x_ref, wq_ref, wk_ref, wv_ref, bq_ref, bk_ref, bv_ref,
                q_ref, k_ref, v_ref, xm_ref):
    xb = x_ref[...]
    q_ref[...] = jnp.dot(xb, wq_ref[...], preferred_element_type=_F32) + bq_ref[...]
    k_ref[...] = jnp.dot(xb, wk_ref[...], preferred_element_type=_F32) + bk_ref[...]
    v_ref[...] = jnp.dot(xb, wv_ref[...], preferred_element_type=_F32) + bv_ref[...]
    xm_ref[...] = jnp.mean(xb, axis=1).reshape(8, 128)


_BN = 1024


@jax.jit
def _dense(xp, wqt, wkt, wvt, bq2, bk2, bv2):
    grid = _NP // _BN
    return pl.pallas_call(
        _dense_body,
        grid=(grid,),
        in_specs=[
            pl.BlockSpec((_BN, _D), lambda i: (i, 0)),
            pl.BlockSpec((_D, _D), lambda i: (0, 0)),
            pl.BlockSpec((_D, _D), lambda i: (0, 0)),
            pl.BlockSpec((_D, _D), lambda i: (0, 0)),
            pl.BlockSpec((1, _D), lambda i: (0, 0)),
            pl.BlockSpec((1, _D), lambda i: (0, 0)),
            pl.BlockSpec((1, _D), lambda i: (0, 0)),
        ],
        out_specs=[
            pl.BlockSpec((_BN, _D), lambda i: (i, 0)),
            pl.BlockSpec((_BN, _D), lambda i: (i, 0)),
            pl.BlockSpec((_BN, _D), lambda i: (i, 0)),
            pl.BlockSpec((_BN // _D, _D), lambda i: (i, 0)),
        ],
        out_shape=[
            jax.ShapeDtypeStruct((_NP, _D), _F32),
            jax.ShapeDtypeStruct((_NP, _D), _F32),
            jax.ShapeDtypeStruct((_NP, _D), _F32),
            jax.ShapeDtypeStruct((_NP // _D, _D), _F32),
        ],
    )(xp, wqt, wkt, wvt, bq2, bk2, bv2)


def _final_body(p0_ref, p1_ref, wot_ref, bo_ref, o_ref):
    acc = p0_ref[...] + p1_ref[...]
    o_ref[...] = jnp.dot(acc, wot_ref[...], preferred_element_type=_F32) + bo_ref[...]


_FBN = 400


@jax.jit
def _final(p0, p1, wot, bo2):
    grid = _N // _FBN
    return pl.pallas_call(
        _final_body,
        grid=(grid,),
        in_specs=[
            pl.BlockSpec((_FBN, _D), lambda i: (i, 0)),
            pl.BlockSpec((_FBN, _D), lambda i: (i, 0)),
            pl.BlockSpec((_D, _D), lambda i: (0, 0)),
            pl.BlockSpec((1, _D), lambda i: (0, 0)),
        ],
        out_specs=pl.BlockSpec((_FBN, _D), lambda i: (i, 0)),
        out_shape=jax.ShapeDtypeStruct((_N, _D), _F32),
    )(p0, p1, wot, bo2)


def _sc_mesh():
    return plsc.VectorSubcoreMesh(
        core_axis_name="c", subcore_axis_name="s",
        num_cores=_NC, num_subcores=_NS)


def _phase_a_fn():
    out_type = (
        jax.ShapeDtypeStruct((_NW, _NCH, _H, _B), _F32),
        jax.ShapeDtypeStruct((_NW, 2, 16), _F32),
    )
    scratch = [
        pltpu.VMEM((_NCH, _B), _I32),
        pltpu.VMEM((_NCH, _B), _I32),
        pltpu.VMEM((_NCH, _B), _F32),
        pltpu.VMEM((_B, _D), _F32),
        pltpu.VMEM((_B, _D), _F32),
        pltpu.VMEM((_B, _D), _F32),
        pltpu.VMEM((_B, _D), _F32),
        pltpu.VMEM((_NP // _D, _D), _F32),
        pltpu.VMEM((2, 16), _F32),
        pltpu.VMEM((_H, _B), _F32),
        pltpu.VMEM((_H, _B), _F32),
        pltpu.VMEM((_H, 16), _F32),
        pltpu.VMEM((_H, 16), _F32),
        pltpu.VMEM((2, 16), _F32),
        pltpu.VMEM((_B,), _I32),
        pltpu.VMEM((_B,), _I32),
        pltpu.VMEM((_B,), _I32),
        pltpu.VMEM((_B,), _I32),
        pltpu.SemaphoreType.DMA,
        pltpu.SemaphoreType.DMA,
        pltpu.SemaphoreType.DMA,
        pltpu.SemaphoreType.DMA,
        pltpu.SemaphoreType.DMA,
        pltpu.SemaphoreType.DMA,
    ]

    @functools.partial(pl.kernel, out_type=out_type, mesh=_sc_mesh(),
                       scratch_types=scratch,
                       compiler_params=pltpu.CompilerParams(
                           needs_layout_passes=False, use_tc_tiling_on_sc=False))
    def phase_a(q_hbm, k_hbm, xm_hbm, row_hbm, col_hbm, ew_hbm, aux_hbm,
                scores_hbm, mz_hbm,
                row_all, col_all, ew_all, qb0, qb1, kb0, kb1, xm_v, aux_v,
                sc0, sc1, m_buf, z_buf, mz_out, ri0, ri1, ci0, ci1,
                sq0, sq1, sk0, sk1, sw0, sw1):
        cid = lax.axis_index("c")
        sid = lax.axis_index("s")
        wid = cid * _NS + sid

        pltpu.sync_copy(row_hbm.at[wid], row_all)
        pltpu.sync_copy(col_hbm.at[wid], col_all)
        pltpu.sync_copy(ew_hbm.at[wid], ew_all)
        pltpu.sync_copy(xm_hbm, xm_v)
        pltpu.sync_copy(aux_hbm, aux_v)

        neg = jnp.full((16,), -1e30, _F32)
        zero = jnp.zeros((16,), _F32)
        for h in range(_H):
            m_buf[h, :] = neg
            z_buf[h, :] = zero
        iota = lax.iota(_I32, 16)
        aux0 = aux_v[0, :]
        aux1 = aux_v[1, :]
        wds = [jnp.full((16,), aux0[h], _F32) for h in range(_H)]
        bds = [jnp.full((16,), aux1[h], _F32) for h in range(_H)]

        def gq(ri, qb, sq):
            return pltpu.make_async_copy(q_hbm.at[ri], qb, sq)

        def gk(ci, kb, sk):
            return pltpu.make_async_copy(k_hbm.at[ci], kb, sk)

        def stage_idx(c, ri, ci):
            for g in range(_NG):
                gs = g * 16
                ri[pl.ds(gs, 16)] = row_all[c, pl.ds(gs, 16)]
                ci[pl.ds(gs, 16)] = col_all[c, pl.ds(gs, 16)]

        def issue(c, ri, ci, qb, kb, sq, sk):
            stage_idx(c, ri, ci)
            gq(ri, qb, sq).start()
            gk(ci, kb, sk).start()

        def wrt(c, scb, sw):
            return pltpu.make_async_copy(scb, scores_hbm.at[wid, c], sw)

        def compute(c, qb, kb, scb):
            def group_body(g, _):
                gs = g * 16
                row16 = row_all[c, pl.ds(gs, 16)]
                col16 = col_all[c, pl.ds(gs, 16)]
                ew16 = ew_all[c, pl.ds(gs, 16)]
                dsc = (plsc.load_gather(
                           xm_v, [lax.shift_right_logical(col16, 7),
                                  lax.bitwise_and(col16, 127)])
                       - plsc.load_gather(
                           xm_v, [lax.shift_right_logical(row16, 7),
                                  lax.bitwise_and(row16, 127)]))
                lidx = iota + gs
                for h in range(_H):
                    acc = dsc * wds[h] + ew16 + bds[h]
                    for d in range(_HD):
                        cix = jnp.full((16,), h * _HD + d, _I32)
                        acc = acc + (plsc.load_gather(qb, [lidx, cix])
                                     * plsc.load_gather(kb, [lidx, cix]))
                    scb[h, pl.ds(gs, 16)] = acc
                    m_old = m_buf[h, :]
                    m_new = jnp.maximum(m_old, acc)
                    z_buf[h, :] = (z_buf[h, :] * jnp.exp(m_old - m_new)
                                   + jnp.exp(acc - m_new))
                    m_buf[h, :] = m_new
                return 0

            lax.fori_loop(0, _NG, group_body, 0)

        issue(0, ri0, ci0, qb0, kb0, sq0, sk0)
        issue(1, ri1, ci1, qb1, kb1, sq1, sk1)
        wrt(0, sc0, sw0).start()
        wrt(1, sc1, sw1).start()

        def pair(j, _):
            c0 = 2 * j
            c1 = c0 + 1
            gq(ri0, qb0, sq0).wait(); gk(ci0, kb0, sk0).wait()
            wrt(c0, sc0, sw0).wait()
            compute(c0, qb0, kb0, sc0)
            wrt(c0, sc0, sw0).start()
            issue(c0 + 2, ri0, ci0, qb0, kb0, sq0, sk0)
            gq(ri1, qb1, sq1).wait(); gk(ci1, kb1, sk1).wait()
            wrt(c1, sc1, sw1).wait()
            compute(c1, qb1, kb1, sc1)
            wrt(c1, sc1, sw1).start()
            c3 = jnp.minimum(c1 + 2, _NCH - 1)
            issue(c3, ri1, ci1, qb1, kb1, sq1, sk1)
            return 0

        lax.fori_loop(0, (_NCH - 1) // 2, pair, 0)
        last = _NCH - 1
        gq(ri0, qb0, sq0).wait(); gk(ci0, kb0, sk0).wait()
        wrt(last, sc0, sw0).wait()
        compute(last, qb0, kb0, sc0)
        wrt(last, sc0, sw0).start()
        gq(ri1, qb1, sq1).wait(); gk(ci1, kb1, sk1).wait()
        wrt(last, sc0, sw0).wait()
        wrt(last - 1, sc1, sw1).wait()

        mvec = jnp.full((16,), -1e30, _F32)
        zvec = jnp.zeros((16,), _F32)
        for h in range(_H):
            mv = m_buf[h, :]
            zv = z_buf[h, :]
            mt = jnp.max(mv)
            zt = jnp.sum(zv * jnp.exp(mv - jnp.full((16,), mt, _F32)))
            sel = iota == h
            mvec = jnp.where(sel, jnp.full((16,), mt, _F32), mvec)
            zvec = jnp.where(sel, jnp.full((16,), zt, _F32), zvec)
        mz_out[0, :] = mvec
        mz_out[1, :] = zvec
        pltpu.sync_copy(mz_out, mz_hbm.at[wid])

    return phase_a


def _phase_b_fn():
    out_type = jax.ShapeDtypeStruct((_NC, 2, _N, _HF), _F32)
    scratch = [
        pltpu.VMEM((_NCH, _B), _I32),
        pltpu.VMEM((_NCH, _B), _I32),
        pltpu.VMEM((_H, _B), _F32),
        pltpu.VMEM((_H, _B), _F32),
        pltpu.VMEM((_B, _HF), _F32),
        pltpu.VMEM((_B, _HF), _F32),
        pltpu.VMEM((_B, _HF), _F32),
        pltpu.VMEM((_B, _HF), _F32),
        pltpu.VMEM((_NW, 2, 16), _F32),
        pltpu.VMEM((_RC, _HF), _F32),
        pltpu.VMEM((_B,), _I32),
        pltpu.VMEM((_B,), _I32),
        pltpu.VMEM((_B,), _I32),
        pltpu.VMEM((_B,), _I32),
        pltpu.VMEM_SHARED((_N, _HF), _F32),
        pltpu.SemaphoreType.DMA,
        pltpu.SemaphoreType.DMA,
        pltpu.SemaphoreType.DMA,
        pltpu.SemaphoreType.DMA,
        pltpu.SemaphoreType.DMA,
        pltpu.SemaphoreType.DMA,
    ]

    @functools.partial(pl.kernel, out_type=out_type, mesh=_sc_mesh(),
                       scratch_types=scratch,
                       compiler_params=pltpu.CompilerParams(
                           needs_layout_passes=False, use_tc_tiling_on_sc=False))
    def phase_b(vlo_hbm, vhi_hbm, scores_hbm, mz_hbm, row_hbm, col_hbm,
                outp_hbm,
                row_all, col_all, sb0, sb1, vb0, vb1, mg0, mg1, mz_all,
                zrow_v, ri0, ri1, ci0, ci1, accum,
                sv0, sv1, sl0, sl1, ss0, ss1):
        cid = lax.axis_index("c")
        sid = lax.axis_index("s")
        wid = cid * _NS + sid

        pltpu.sync_copy(row_hbm.at[wid], row_all)
        pltpu.sync_copy(col_hbm.at[wid], col_all)

        pltpu.sync_copy(mz_hbm, mz_all)
        gm = mz_all[0, 0, :]
        for t in range(1, _NW):
            gm = jnp.maximum(gm, mz_all[t, 0, :])
        gz = jnp.zeros((16,), _F32)
        for t in range(_NW):
            gz = gz + mz_all[t, 1, :] * jnp.exp(mz_all[t, 0, :] - gm)
        ginv = 1.0 / gz
        gms = [jnp.full((16,), gm[h], _F32) for h in range(_H)]
        gis = [jnp.full((16,), ginv[h], _F32) for h in range(_H)]

        z16 = jnp.zeros((16,), _F32)

        def zr_body(r, _):
            for j in range(_HF // 16):
                zrow_v[r, pl.ds(j * 16, 16)] = z16
            return 0

        lax.fori_loop(0, _RC, zr_body, 0)

        def gsc(c, sb, sl):
            return pltpu.make_async_copy(scores_hbm.at[wid, c], sb, sl)

        def one_pass(half, v_hbm):
            heads = [half * _HH + j for j in range(_HH)]

            def zmsg_body(r, _):
                for j in range(_HF // 16):
                    mg0[r, pl.ds(j * 16, 16)] = z16
                    mg1[r, pl.ds(j * 16, 16)] = z16
                return 0

            lax.fori_loop(0, _B, zmsg_body, 0)
            for j in range(_RPT // _RC):
                pltpu.sync_copy(zrow_v,
                                accum.at[pl.ds(sid * _RPT + j * _RC, _RC)])
            plsc.subcore_barrier()

            def gv(ci, vb, sv):
                return pltpu.make_async_copy(v_hbm.at[ci], vb, sv)

            def sct(ri, mg, ss):
                return pltpu.make_async_copy(mg, accum.at[ri], ss)

            def issue_gv(c, ci, vb, sv):
                for g in range(_NG):
                    gs = g * 16
                    ci[pl.ds(gs, 16)] = col_all[c, pl.ds(gs, 16)]
                gv(ci, vb, sv).start()

            def issue_sct(c, ri, mg, ss):
                for g in range(_NG):
                    gs = g * 16
                    ri[pl.ds(gs, 16)] = row_all[c, pl.ds(gs, 16)]
                sct(ri, mg, ss).start(add=True)

            def compute(c, vb, sb, mg):
                def group_body(g, _):
                    gs = g * 16
                    attns = []
                    for h in heads:
                        s = sb[h, pl.ds(gs, 16)]
                        attns.append(jnp.exp(s - gms[h]) * gis[h])
                    for b in range(16):
                        e = gs + b
                        for j in range(_HH):
                            asc = jnp.full((16,), attns[j][b], _F32)
                            mg[e, pl.ds(j * _HD, _HD)] = (
                                vb[e, pl.ds(j * _HD, _HD)] * asc)
                    return 0

                lax.fori_loop(0, _NG, group_body, 0)

            issue_sct(0, ri0, mg0, ss0)
            issue_sct(1, ri1, mg1, ss1)
            issue_gv(0, ci0, vb0, sv0); gsc(0, sb0, sl0).start()
            issue_gv(1, ci1, vb1, sv1); gsc(1, sb1, sl1).start()

            def pair(j, _):
                c0 = 2 * j
                c1 = c0 + 1
                gv(ci0, vb0, sv0).wait(); gsc(c0, sb0, sl0).wait()
                sct(ri0, mg0, ss0).wait()
                compute(c0, vb0, sb0, mg0)
                issue_sct(c0, ri0, mg0, ss0)
                issue_gv(c0 + 2, ci0, vb0, sv0); gsc(c0 + 2, sb0, sl0).start()
                gv(ci1, vb1, sv1).wait(); gsc(c1, sb1, sl1).wait()
                sct(ri1, mg1, ss1).wait()
                compute(c1, vb1, sb1, mg1)
                issue_sct(c1, ri1, mg1, ss1)
                c3 = jnp.minimum(c1 + 2, _NCH - 1)
                issue_gv(c3, ci1, vb1, sv1); gsc(c3, sb1, sl1).start()
                return 0

            lax.fori_loop(0, (_NCH - 1) // 2, pair, 0)
            last = _NCH - 1
            gv(ci0, vb0, sv0).wait(); gsc(last, sb0, sl0).wait()
            sct(ri0, mg0, ss0).wait()
            compute(last, vb0, sb0, mg0)
            issue_sct(last, ri0, mg0, ss0)
            gv(ci1, vb1, sv1).wait(); gsc(last, sb1, sl1).wait()
            sct(ri0, mg0, ss0).wait()
            sct(ri1, mg1, ss1).wait()
            plsc.subcore_barrier()

            for j in range(_RPT // _RC):
                rb = sid * _RPT + j * _RC
                pltpu.sync_copy(accum.at[pl.ds(rb, _RC)],
                                outp_hbm.at[cid, half, pl.ds(rb, _RC)])

        one_pass(0, vlo_hbm)
        one_pass(1, vhi_hbm)

    return phase_b


_PHASE_A = _phase_a_fn()
_PHASE_B = _phase_b_fn()


@jax.jit
def _run(x, edge_index, edge_weight, Wq, bq, Wk, bk, Wv, bv, Wd, bd, Wo, bo):
    scale = 1.0 / math.sqrt(_HD)
    xp = jnp.pad(x, ((0, _NP - _N), (0, 0)))
    qs, ks, vs, xm = _dense(
        xp,
        Wq.T * scale, Wk.T, Wv.T,
        (bq * scale).reshape(1, _D), bk.reshape(1, _D), bv.reshape(1, _D),
    )
    row = edge_index[0].reshape(_NW, _NCH, _B)
    col = edge_index[1].reshape(_NW, _NCH, _B)
    ew3 = edge_weight.reshape(_NW, _NCH, _B)
    aux = (jnp.zeros((2, 16), _F32)
           .at[0, :_H].set(Wd[:, 0])
           .at[1, :_H].set(bd))
    scores, mz = _PHASE_A(qs, ks, xm, row, col, ew3, aux)
    partials = _PHASE_B(vs[:, :_HF], vs[:, _HF:], scores, mz, row, col)
    p = jnp.concatenate([partials[:, 0], partials[:, 1]], axis=-1)
    return _final(p[0], p[1], Wo.T, bo.reshape(1, _D))


def kernel(x, edge_index, edge_weight, Wq, bq, Wk, bk, Wv, bv, Wd, bd, Wo, bo):
    return _run(x, edge_index, edge_weight, Wq, bq, Wk, bk, Wv, bv,
                Wd, bd, Wo, bo)


# speedup vs baseline: 15.9619x; 2.8538x over previous
import functools
import math

import jax
import jax.numpy as jnp
from jax import lax
from jax.experimental import pallas as pl
from jax.experimental.pallas import tpu as pltpu
from jax.experimental.pallas import tpu_sc as plsc

_N = 10000
_NP = 10240
_E = 320000
_D = 128
_H = 8
_HD = 16
_NC = 2
_NS = 16
_NW = _NC * _NS
_EPT = _E // _NW
_B = 80
_NCH = _EPT // _B
_NG = _B // 16
_RPT = _N // _NS
_RC = 125
_HF = _D // 2
_DP = _D + 1
_HH = _H // 2

_F32 = jnp.float32
_I32 = jnp.int32


def _swizzle(m):
    rows = jax.lax.broadcasted_iota(jnp.int32, (m.shape[0], 1), 0) % 16
    out = m
    for r in range(1, 16):
        rolled = jnp.concatenate([m[:, -r:], m[:, :-r]], axis=1)
        out = jnp.where(rows == r, rolled, out)
    return out


def _dense_body(x_ref, wq_ref, wk_ref, wv_ref, bq_ref, bk_ref, bv_ref,
                q_ref, k_ref, v_ref, xm_ref):
    xb = x_ref[...]
    q_ref[...] = _swizzle(
        jnp.dot(xb, wq_ref[...], preferred_element_type=_F32) + bq_ref[...])
    k_ref[...] = _swizzle(
        jnp.dot(xb, wk_ref[...], preferred_element_type=_F32) + bk_ref[...])
    v_ref[...] = jnp.dot(xb, wv_ref[...], preferred_element_type=_F32) + bv_ref[...]
    xm_ref[...] = jnp.mean(xb, axis=1).reshape(8, 128)


_BN = 1024


@jax.jit
def _dense(xp, wqt, wkt, wvt, bq2, bk2, bv2):
    grid = _NP // _BN
    return pl.pallas_call(
        _dense_body,
        grid=(grid,),
        in_specs=[
            pl.BlockSpec((_BN, _D), lambda i: (i, 0)),
            pl.BlockSpec((_D, _D), lambda i: (0, 0)),
            pl.BlockSpec((_D, _D), lambda i: (0, 0)),
            pl.BlockSpec((_D, _D), lambda i: (0, 0)),
            pl.BlockSpec((1, _D), lambda i: (0, 0)),
            pl.BlockSpec((1, _D), lambda i: (0, 0)),
            pl.BlockSpec((1, _D), lambda i: (0, 0)),
        ],
        out_specs=[
            pl.BlockSpec((_BN, _D), lambda i: (i, 0)),
            pl.BlockSpec((_BN, _D), lambda i: (i, 0)),
            pl.BlockSpec((_BN, _D), lambda i: (i, 0)),
            pl.BlockSpec((_BN // _D, _D), lambda i: (i, 0)),
        ],
        out_shape=[
            jax.ShapeDtypeStruct((_NP, _D), _F32),
            jax.ShapeDtypeStruct((_NP, _D), _F32),
            jax.ShapeDtypeStruct((_NP, _D), _F32),
            jax.ShapeDtypeStruct((_NP // _D, _D), _F32),
        ],
    )(xp, wqt, wkt, wvt, bq2, bk2, bv2)


def _final_body(p0_ref, p1_ref, wot_ref, bo_ref, o_ref):
    acc = p0_ref[...] + p1_ref[...]
    o_ref[...] = jnp.dot(acc, wot_ref[...], preferred_element_type=_F32) + bo_ref[...]


_FBN = 400


@jax.jit
def _final(p0, p1, wot, bo2):
    grid = _N // _FBN
    return pl.pallas_call(
        _final_body,
        grid=(grid,),
        in_specs=[
            pl.BlockSpec((_FBN, _D), lambda i: (i, 0)),
            pl.BlockSpec((_FBN, _D), lambda i: (i, 0)),
            pl.BlockSpec((_D, _D), lambda i: (0, 0)),
            pl.BlockSpec((1, _D), lambda i: (0, 0)),
        ],
        out_specs=pl.BlockSpec((_FBN, _D), lambda i: (i, 0)),
        out_shape=jax.ShapeDtypeStruct((_N, _D), _F32),
    )(p0, p1, wot, bo2)


def _sc_mesh():
    return plsc.VectorSubcoreMesh(
        core_axis_name="c", subcore_axis_name="s",
        num_cores=_NC, num_subcores=_NS)


def _phase_a_fn():
    out_type = (
        jax.ShapeDtypeStruct((_NW, _NCH, _H, _B), _F32),
        jax.ShapeDtypeStruct((_NW, 2, 16), _F32),
    )
    scratch = [
        pltpu.VMEM((_NCH, _B), _I32),
        pltpu.VMEM((_NCH, _B), _I32),
        pltpu.VMEM((_NCH, _B), _F32),
        pltpu.VMEM((_B, _D), _F32),
        pltpu.VMEM((_B, _D), _F32),
        pltpu.VMEM((_B, _D), _F32),
        pltpu.VMEM((_B, _D), _F32),
        pltpu.VMEM((_NP // _D, _D), _F32),
        pltpu.VMEM((2, 16), _F32),
        pltpu.VMEM((_H, _B), _F32),
        pltpu.VMEM((_H, _B), _F32),
        pltpu.VMEM((_H, 16), _F32),
        pltpu.VMEM((_H, 16), _F32),
        pltpu.VMEM((2, 16), _F32),
        pltpu.VMEM((_B,), _I32),
        pltpu.VMEM((_B,), _I32),
        pltpu.VMEM((_B,), _I32),
        pltpu.VMEM((_B,), _I32),
        pltpu.SemaphoreType.DMA,
        pltpu.SemaphoreType.DMA,
        pltpu.SemaphoreType.DMA,
        pltpu.SemaphoreType.DMA,
        pltpu.SemaphoreType.DMA,
        pltpu.SemaphoreType.DMA,
    ]

    @functools.partial(pl.kernel, out_type=out_type, mesh=_sc_mesh(),
                       scratch_types=scratch,
                       compiler_params=pltpu.CompilerParams(
                           needs_layout_passes=False, use_tc_tiling_on_sc=False))
    def phase_a(q_hbm, k_hbm, xm_hbm, row_hbm, col_hbm, ew_hbm, aux_hbm,
                scores_hbm, mz_hbm,
                row_all, col_all, ew_all, qb0, qb1, kb0, kb1, xm_v, aux_v,
                sc0, sc1, m_buf, z_buf, mz_out, ri0, ri1, ci0, ci1,
                sq0, sq1, sk0, sk1, sw0, sw1):
        cid = lax.axis_index("c")
        sid = lax.axis_index("s")
        wid = cid * _NS + sid

        pltpu.sync_copy(row_hbm.at[wid], row_all)
        pltpu.sync_copy(col_hbm.at[wid], col_all)
        pltpu.sync_copy(ew_hbm.at[wid], ew_all)
        pltpu.sync_copy(xm_hbm, xm_v)
        pltpu.sync_copy(aux_hbm, aux_v)

        neg = jnp.full((16,), -1e30, _F32)
        zero = jnp.zeros((16,), _F32)
        for h in range(_H):
            m_buf[h, :] = neg
            z_buf[h, :] = zero
        iota = lax.iota(_I32, 16)
        aux0 = aux_v[0, :]
        aux1 = aux_v[1, :]
        wds = [jnp.full((16,), aux0[h], _F32) for h in range(_H)]
        bds = [jnp.full((16,), aux1[h], _F32) for h in range(_H)]

        def gq(ri, qb, sq):
            return pltpu.make_async_copy(q_hbm.at[ri], qb, sq)

        def gk(ci, kb, sk):
            return pltpu.make_async_copy(k_hbm.at[ci], kb, sk)

        def stage_idx(c, ri, ci):
            for g in range(_NG):
                gs = g * 16
                ri[pl.ds(gs, 16)] = row_all[c, pl.ds(gs, 16)]
                ci[pl.ds(gs, 16)] = col_all[c, pl.ds(gs, 16)]

        def issue(c, ri, ci, qb, kb, sq, sk):
            stage_idx(c, ri, ci)
            gq(ri, qb, sq).start()
            gk(ci, kb, sk).start()

        def wrt(c, scb, sw):
            return pltpu.make_async_copy(scb, scores_hbm.at[wid, c], sw)

        def compute(c, qb, kb, scb):
            def group_body(g, _):
                gs = g * 16
                row16 = row_all[c, pl.ds(gs, 16)]
                col16 = col_all[c, pl.ds(gs, 16)]
                ew16 = ew_all[c, pl.ds(gs, 16)]
                dsc = (plsc.load_gather(
                           xm_v, [lax.shift_right_logical(col16, 7),
                                  lax.bitwise_and(col16, 127)])
                       - plsc.load_gather(
                           xm_v, [lax.shift_right_logical(row16, 7),
                                  lax.bitwise_and(row16, 127)]))
                lidx = iota + gs
                rotq = lax.bitwise_and(row16, 15)
                rotk = lax.bitwise_and(col16, 15)
                for h in range(_H):
                    acc = dsc * wds[h] + ew16 + bds[h]
                    for d in range(_HD):
                        hd = h * _HD + d
                        cq = lax.bitwise_and(rotq + hd, 127)
                        ck = lax.bitwise_and(rotk + hd, 127)
                        acc = acc + (plsc.load_gather(qb, [lidx, cq])
                                     * plsc.load_gather(kb, [lidx, ck]))
                    scb[h, pl.ds(gs, 16)] = acc
                    m_old = m_buf[h, :]
                    m_new = jnp.maximum(m_old, acc)
                    z_buf[h, :] = (z_buf[h, :] * jnp.exp(m_old - m_new)
                                   + jnp.exp(acc - m_new))
                    m_buf[h, :] = m_new
                return 0

            lax.fori_loop(0, _NG, group_body, 0)

        issue(0, ri0, ci0, qb0, kb0, sq0, sk0)
        issue(1, ri1, ci1, qb1, kb1, sq1, sk1)
        wrt(0, sc0, sw0).start()
        wrt(1, sc1, sw1).start()

        def pair(j, _):
            c0 = 2 * j
            c1 = c0 + 1
            gq(ri0, qb0, sq0).wait(); gk(ci0, kb0, sk0).wait()
            wrt(c0, sc0, sw0).wait()
            compute(c0, qb0, kb0, sc0)
            wrt(c0, sc0, sw0).start()
            issue(c0 + 2, ri0, ci0, qb0, kb0, sq0, sk0)
            gq(ri1, qb1, sq1).wait(); gk(ci1, kb1, sk1).wait()
            wrt(c1, sc1, sw1).wait()
            compute(c1, qb1, kb1, sc1)
            wrt(c1, sc1, sw1).start()
            c3 = jnp.minimum(c1 + 2, _NCH - 1)
            issue(c3, ri1, ci1, qb1, kb1, sq1, sk1)
            return 0

        lax.fori_loop(0, (_NCH - 1) // 2, pair, 0)
        last = _NCH - 1
        gq(ri0, qb0, sq0).wait(); gk(ci0, kb0, sk0).wait()
        wrt(last, sc0, sw0).wait()
        compute(last, qb0, kb0, sc0)
        wrt(last, sc0, sw0).start()
        gq(ri1, qb1, sq1).wait(); gk(ci1, kb1, sk1).wait()
        wrt(last, sc0, sw0).wait()
        wrt(last - 1, sc1, sw1).wait()

        mvec = jnp.full((16,), -1e30, _F32)
        zvec = jnp.zeros((16,), _F32)
        for h in range(_H):
            mv = m_buf[h, :]
            zv = z_buf[h, :]
            mt = jnp.max(mv)
            zt = jnp.sum(zv * jnp.exp(mv - jnp.full((16,), mt, _F32)))
            sel = iota == h
            mvec = jnp.where(sel, jnp.full((16,), mt, _F32), mvec)
            zvec = jnp.where(sel, jnp.full((16,), zt, _F32), zvec)
        mz_out[0, :] = mvec
        mz_out[1, :] = zvec
        pltpu.sync_copy(mz_out, mz_hbm.at[wid])

    return phase_a


def _phase_b_fn():
    out_type = jax.ShapeDtypeStruct((_NC, 2, _N, _HF), _F32)
    scratch = [
        pltpu.VMEM((_NCH, _B), _I32),
        pltpu.VMEM((_NCH, _B), _I32),
        pltpu.VMEM((_H, _B), _F32),
        pltpu.VMEM((_H, _B), _F32),
        pltpu.VMEM((_B, _HF), _F32),
        pltpu.VMEM((_B, _HF), _F32),
        pltpu.VMEM((_B, _HF), _F32),
        pltpu.VMEM((_B, _HF), _F32),
        pltpu.VMEM((_NW, 2, 16), _F32),
        pltpu.VMEM((_RC, _HF), _F32),
        pltpu.VMEM((_B,), _I32),
        pltpu.VMEM((_B,), _I32),
        pltpu.VMEM((_B,), _I32),
        pltpu.VMEM((_B,), _I32),
        pltpu.VMEM_SHARED((_N, _HF), _F32),
        pltpu.SemaphoreType.DMA,
        pltpu.SemaphoreType.DMA,
        pltpu.SemaphoreType.DMA,
        pltpu.SemaphoreType.DMA,
        pltpu.SemaphoreType.DMA,
        pltpu.SemaphoreType.DMA,
    ]

    @functools.partial(pl.kernel, out_type=out_type, mesh=_sc_mesh(),
                       scratch_types=scratch,
                       compiler_params=pltpu.CompilerParams(
                           needs_layout_passes=False, use_tc_tiling_on_sc=False))
    def phase_b(vlo_hbm, vhi_hbm, scores_hbm, mz_hbm, row_hbm, col_hbm,
                outp_hbm,
                row_all, col_all, sb0, sb1, vb0, vb1, mg0, mg1, mz_all,
                zrow_v, ri0, ri1, ci0, ci1, accum,
                sv0, sv1, sl0, sl1, ss0, ss1):
        cid = lax.axis_index("c")
        sid = lax.axis_index("s")
        wid = cid * _NS + sid

        pltpu.sync_copy(row_hbm.at[wid], row_all)
        pltpu.sync_copy(col_hbm.at[wid], col_all)

        pltpu.sync_copy(mz_hbm, mz_all)
        gm = mz_all[0, 0, :]
        for t in range(1, _NW):
            gm = jnp.maximum(gm, mz_all[t, 0, :])
        gz = jnp.zeros((16,), _F32)
        for t in range(_NW):
            gz = gz + mz_all[t, 1, :] * jnp.exp(mz_all[t, 0, :] - gm)
        ginv = 1.0 / gz
        gms = [jnp.full((16,), gm[h], _F32) for h in range(_H)]
        gis = [jnp.full((16,), ginv[h], _F32) for h in range(_H)]

        z16 = jnp.zeros((16,), _F32)

        def zr_body(r, _):
            for j in range(_HF // 16):
                zrow_v[r, pl.ds(j * 16, 16)] = z16
            return 0

        lax.fori_loop(0, _RC, zr_body, 0)

        def gsc(c, sb, sl):
            return pltpu.make_async_copy(scores_hbm.at[wid, c], sb, sl)

        def one_pass(half, v_hbm):
            heads = [half * _HH + j for j in range(_HH)]

            def zmsg_body(r, _):
                for j in range(_HF // 16):
                    mg0[r, pl.ds(j * 16, 16)] = z16
                    mg1[r, pl.ds(j * 16, 16)] = z16
                return 0

            lax.fori_loop(0, _B, zmsg_body, 0)
            for j in range(_RPT // _RC):
                pltpu.sync_copy(zrow_v,
                                accum.at[pl.ds(sid * _RPT + j * _RC, _RC)])
            plsc.subcore_barrier()

            def gv(ci, vb, sv):
                return pltpu.make_async_copy(v_hbm.at[ci], vb, sv)

            def sct(ri, mg, ss):
                return pltpu.make_async_copy(mg, accum.at[ri], ss)

            def issue_gv(c, ci, vb, sv):
                for g in range(_NG):
                    gs = g * 16
                    ci[pl.ds(gs, 16)] = col_all[c, pl.ds(gs, 16)]
                gv(ci, vb, sv).start()

            def issue_sct(c, ri, mg, ss):
                for g in range(_NG):
                    gs = g * 16
                    ri[pl.ds(gs, 16)] = row_all[c, pl.ds(gs, 16)]
                sct(ri, mg, ss).start(add=True)

            def compute(c, vb, sb, mg):
                def group_body(g, _):
                    gs = g * 16
                    attns = []
                    for h in heads:
                        s = sb[h, pl.ds(gs, 16)]
                        attns.append(jnp.exp(s - gms[h]) * gis[h])
                    for b in range(16):
                        e = gs + b
                        for j in range(_HH):
                            asc = jnp.full((16,), attns[j][b], _F32)
                            mg[e, pl.ds(j * _HD, _HD)] = (
                                vb[e, pl.ds(j * _HD, _HD)] * asc)
                    return 0

                lax.fori_loop(0, _NG, group_body, 0)

            issue_sct(0, ri0, mg0, ss0)
            issue_sct(1, ri1, mg1, ss1)
            issue_gv(0, ci0, vb0, sv0); gsc(0, sb0, sl0).start()
            issue_gv(1, ci1, vb1, sv1); gsc(1, sb1, sl1).start()

            def pair(j, _):
                c0 = 2 * j
                c1 = c0 + 1
                gv(ci0, vb0, sv0).wait(); gsc(c0, sb0, sl0).wait()
                sct(ri0, mg0, ss0).wait()
                compute(c0, vb0, sb0, mg0)
                issue_sct(c0, ri0, mg0, ss0)
                issue_gv(c0 + 2, ci0, vb0, sv0); gsc(c0 + 2, sb0, sl0).start()
                gv(ci1, vb1, sv1).wait(); gsc(c1, sb1, sl1).wait()
                sct(ri1, mg1, ss1).wait()
                compute(c1, vb1, sb1, mg1)
                issue_sct(c1, ri1, mg1, ss1)
                c3 = jnp.minimum(c1 + 2, _NCH - 1)
                issue_gv(c3, ci1, vb1, sv1); gsc(c3, sb1, sl1).start()
                return 0

            lax.fori_loop(0, (_NCH - 1) // 2, pair, 0)
            last = _NCH - 1
            gv(ci0, vb0, sv0).wait(); gsc(last, sb0, sl0).wait()
            sct(ri0, mg0, ss0).wait()
            compute(last, vb0, sb0, mg0)
            issue_sct(last, ri0, mg0, ss0)
            gv(ci1, vb1, sv1).wait(); gsc(last, sb1, sl1).wait()
            sct(ri0, mg0, ss0).wait()
            sct(ri1, mg1, ss1).wait()
            plsc.subcore_barrier()

            for j in range(_RPT // _RC):
                rb = sid * _RPT + j * _RC
                pltpu.sync_copy(accum.at[pl.ds(rb, _RC)],
                                outp_hbm.at[cid, half, pl.ds(rb, _RC)])

        one_pass(0, vlo_hbm)
        one_pass(1, vhi_hbm)

    return phase_b


_PHASE_A = _phase_a_fn()
_PHASE_B = _phase_b_fn()


@jax.jit
def _run(x, edge_index, edge_weight, Wq, bq, Wk, bk, Wv, bv, Wd, bd, Wo, bo):
    scale = 1.0 / math.sqrt(_HD)
    xp = jnp.pad(x, ((0, _NP - _N), (0, 0)))
    qs, ks, vs, xm = _dense(
        xp,
        Wq.T * scale, Wk.T, Wv.T,
        (bq * scale).reshape(1, _D), bk.reshape(1, _D), bv.reshape(1, _D),
    )
    row = edge_index[0].reshape(_NW, _NCH, _B)
    col = edge_index[1].reshape(_NW, _NCH, _B)
    ew3 = edge_weight.reshape(_NW, _NCH, _B)
    aux = (jnp.zeros((2, 16), _F32)
           .at[0, :_H].set(Wd[:, 0])
           .at[1, :_H].set(bd))
    scores, mz = _PHASE_A(qs, ks, xm, row, col, ew3, aux)
    partials = _PHASE_B(vs[:, :_HF], vs[:, _HF:], scores, mz, row, col)
    p = jnp.concatenate([partials[:, 0], partials[:, 1]], axis=-1)
    return _final(p[0], p[1], Wo.T, bo.reshape(1, _D))


def kernel(x, edge_index, edge_weight, Wq, bq, Wk, bk, Wv, bv, Wd, bd, Wo, bo):
    return _run(x, edge_index, edge_weight, Wq, bq, Wk, bk, Wv, bv,
                Wd, bd, Wo, bo)


# speedup vs baseline: 17.7207x; 1.1102x over previous
import functools
import math

import jax
import jax.numpy as jnp
from jax import lax
from jax.experimental import pallas as pl
from jax.experimental.pallas import tpu as pltpu
from jax.experimental.pallas import tpu_sc as plsc

_N = 10000
_NP = 10240
_E = 320000
_D = 128
_H = 8
_HD = 16
_NC = 2
_NS = 16
_NW = _NC * _NS
_EPT = _E // _NW
_B = 80
_NCH = _EPT // _B
_NG = _B // 16
_RPT = _N // _NS
_RC = 125
_HF = _D // 2
_DP = _D + 1
_HH = _H // 2

_F32 = jnp.float32
_I32 = jnp.int32


def _dense_body(x_ref, wq_ref, wk_ref, wv_ref, bq_ref, bk_ref, bv_ref,
                q_ref, k_ref, v_ref, xm_ref):
    xb = x_ref[...]
    q_ref[...] = (
        jnp.dot(xb, wq_ref[...], preferred_element_type=_F32) + bq_ref[...])
    k_ref[...] = (
        jnp.dot(xb, wk_ref[...], preferred_element_type=_F32) + bk_ref[...])
    v_ref[...] = jnp.dot(xb, wv_ref[...], preferred_element_type=_F32) + bv_ref[...]
    xm_ref[...] = jnp.mean(xb, axis=1).reshape(8, 128)


_BN = 1024


@jax.jit
def _dense(xp, wqt, wkt, wvt, bq2, bk2, bv2):
    grid = _NP // _BN
    return pl.pallas_call(
        _dense_body,
        grid=(grid,),
        in_specs=[
            pl.BlockSpec((_BN, _D), lambda i: (i, 0)),
            pl.BlockSpec((_D, _D), lambda i: (0, 0)),
            pl.BlockSpec((_D, _D), lambda i: (0, 0)),
            pl.BlockSpec((_D, _D), lambda i: (0, 0)),
            pl.BlockSpec((1, _D), lambda i: (0, 0)),
            pl.BlockSpec((1, _D), lambda i: (0, 0)),
            pl.BlockSpec((1, _D), lambda i: (0, 0)),
        ],
        out_specs=[
            pl.BlockSpec((_BN, _D), lambda i: (i, 0)),
            pl.BlockSpec((_BN, _D), lambda i: (i, 0)),
            pl.BlockSpec((_BN, _D), lambda i: (i, 0)),
            pl.BlockSpec((_BN // _D, _D), lambda i: (i, 0)),
        ],
        out_shape=[
            jax.ShapeDtypeStruct((_NP, _D), _F32),
            jax.ShapeDtypeStruct((_NP, _D), _F32),
            jax.ShapeDtypeStruct((_NP, _D), _F32),
            jax.ShapeDtypeStruct((_NP // _D, _D), _F32),
        ],
    )(xp, wqt, wkt, wvt, bq2, bk2, bv2)


def _final_body(p0_ref, p1_ref, wot_ref, bo_ref, o_ref):
    acc = p0_ref[...] + p1_ref[...]
    o_ref[...] = jnp.dot(acc, wot_ref[...], preferred_element_type=_F32) + bo_ref[...]


_FBN = 400


@jax.jit
def _final(p0, p1, wot, bo2):
    grid = _N // _FBN
    return pl.pallas_call(
        _final_body,
        grid=(grid,),
        in_specs=[
            pl.BlockSpec((_FBN, _D), lambda i: (i, 0)),
            pl.BlockSpec((_FBN, _D), lambda i: (i, 0)),
            pl.BlockSpec((_D, _D), lambda i: (0, 0)),
            pl.BlockSpec((1, _D), lambda i: (0, 0)),
        ],
        out_specs=pl.BlockSpec((_FBN, _D), lambda i: (i, 0)),
        out_shape=jax.ShapeDtypeStruct((_N, _D), _F32),
    )(p0, p1, wot, bo2)


def _sc_mesh():
    return plsc.VectorSubcoreMesh(
        core_axis_name="c", subcore_axis_name="s",
        num_cores=_NC, num_subcores=_NS)


def _phase_a_fn():
    out_type = (
        jax.ShapeDtypeStruct((_NW, _NCH, _H, _B), _F32),
        jax.ShapeDtypeStruct((_NW, 2, 16), _F32),
    )
    scratch = [
        pltpu.VMEM((_NCH, _B), _I32),
        pltpu.VMEM((_NCH, _B), _I32),
        pltpu.VMEM((_NCH, _B), _F32),
        pltpu.VMEM((_B, _D), _F32),
        pltpu.VMEM((_B, _D), _F32),
        pltpu.VMEM((_B, _D), _F32),
        pltpu.VMEM((_B, _D), _F32),
        pltpu.VMEM((_NP // _D, _D), _F32),
        pltpu.VMEM((2, 16), _F32),
        pltpu.VMEM((_H, _B), _F32),
        pltpu.VMEM((_H, _B), _F32),
        pltpu.VMEM((_H, 16), _F32),
        pltpu.VMEM((_H, 16), _F32),
        pltpu.VMEM((2, 16), _F32),
        pltpu.VMEM((_B,), _I32),
        pltpu.VMEM((_B,), _I32),
        pltpu.VMEM((_B,), _I32),
        pltpu.VMEM((_B,), _I32),
        pltpu.SemaphoreType.DMA,
        pltpu.SemaphoreType.DMA,
        pltpu.SemaphoreType.DMA,
        pltpu.SemaphoreType.DMA,
        pltpu.SemaphoreType.DMA,
        pltpu.SemaphoreType.DMA,
    ]

    @functools.partial(pl.kernel, out_type=out_type, mesh=_sc_mesh(),
                       scratch_types=scratch,
                       compiler_params=pltpu.CompilerParams(
                           needs_layout_passes=False, use_tc_tiling_on_sc=False))
    def phase_a(q_hbm, k_hbm, xm_hbm, row_hbm, col_hbm, ew_hbm, aux_hbm,
                scores_hbm, mz_hbm,
                row_all, col_all, ew_all, qb0, qb1, kb0, kb1, xm_v, aux_v,
                sc0, sc1, m_buf, z_buf, mz_out, ri0, ri1, ci0, ci1,
                sq0, sq1, sk0, sk1, sw0, sw1):
        cid = lax.axis_index("c")
        sid = lax.axis_index("s")
        wid = cid * _NS + sid

        pltpu.sync_copy(row_hbm.at[wid], row_all)
        pltpu.sync_copy(col_hbm.at[wid], col_all)
        pltpu.sync_copy(ew_hbm.at[wid], ew_all)
        pltpu.sync_copy(xm_hbm, xm_v)
        pltpu.sync_copy(aux_hbm, aux_v)

        neg = jnp.full((16,), -1e30, _F32)
        zero = jnp.zeros((16,), _F32)
        for h in range(_H):
            m_buf[h, :] = neg
            z_buf[h, :] = zero
        iota = lax.iota(_I32, 16)
        aux0 = aux_v[0, :]
        aux1 = aux_v[1, :]
        wds = [jnp.full((16,), aux0[h], _F32) for h in range(_H)]
        bds = [jnp.full((16,), aux1[h], _F32) for h in range(_H)]
        diags = [lax.bitwise_and(iota + d, 15) for d in range(_HD)]

        def gq(ri, qb, sq):
            return pltpu.make_async_copy(q_hbm.at[ri], qb, sq)

        def gk(ci, kb, sk):
            return pltpu.make_async_copy(k_hbm.at[ci], kb, sk)

        def stage_idx(c, ri, ci):
            for g in range(_NG):
                gs = g * 16
                ri[pl.ds(gs, 16)] = row_all[c, pl.ds(gs, 16)]
                ci[pl.ds(gs, 16)] = col_all[c, pl.ds(gs, 16)]

        def issue(c, ri, ci, qb, kb, sq, sk):
            stage_idx(c, ri, ci)
            gq(ri, qb, sq).start()
            gk(ci, kb, sk).start()

        def wrt(c, scb, sw):
            return pltpu.make_async_copy(scb, scores_hbm.at[wid, c], sw)

        def compute(c, qb, kb, scb):
            def group_body(g, _):
                gs = g * 16
                row16 = row_all[c, pl.ds(gs, 16)]
                col16 = col_all[c, pl.ds(gs, 16)]
                ew16 = ew_all[c, pl.ds(gs, 16)]
                dsc = (plsc.load_gather(
                           xm_v, [lax.shift_right_logical(col16, 7),
                                  lax.bitwise_and(col16, 127)])
                       - plsc.load_gather(
                           xm_v, [lax.shift_right_logical(row16, 7),
                                  lax.bitwise_and(row16, 127)]))
                lidx = iota + gs
                for h in range(_H):
                    acc = dsc * wds[h] + ew16 + bds[h]
                    for d in range(_HD):
                        cix = diags[d] + (h * _HD)
                        acc = acc + (plsc.load_gather(qb, [lidx, cix])
                                     * plsc.load_gather(kb, [lidx, cix]))
                    scb[h, pl.ds(gs, 16)] = acc
                    m_old = m_buf[h, :]
                    m_new = jnp.maximum(m_old, acc)
                    z_buf[h, :] = (z_buf[h, :] * jnp.exp(m_old - m_new)
                                   + jnp.exp(acc - m_new))
                    m_buf[h, :] = m_new
                return 0

            lax.fori_loop(0, _NG, group_body, 0)

        issue(0, ri0, ci0, qb0, kb0, sq0, sk0)
        issue(1, ri1, ci1, qb1, kb1, sq1, sk1)
        wrt(0, sc0, sw0).start()
        wrt(1, sc1, sw1).start()

        def pair(j, _):
            c0 = 2 * j
            c1 = c0 + 1
            gq(ri0, qb0, sq0).wait(); gk(ci0, kb0, sk0).wait()
            wrt(c0, sc0, sw0).wait()
            compute(c0, qb0, kb0, sc0)
            wrt(c0, sc0, sw0).start()
            issue(c0 + 2, ri0, ci0, qb0, kb0, sq0, sk0)
            gq(ri1, qb1, sq1).wait(); gk(ci1, kb1, sk1).wait()
            wrt(c1, sc1, sw1).wait()
            compute(c1, qb1, kb1, sc1)
            wrt(c1, sc1, sw1).start()
            c3 = jnp.minimum(c1 + 2, _NCH - 1)
            issue(c3, ri1, ci1, qb1, kb1, sq1, sk1)
            return 0

        lax.fori_loop(0, (_NCH - 1) // 2, pair, 0)
        last = _NCH - 1
        gq(ri0, qb0, sq0).wait(); gk(ci0, kb0, sk0).wait()
        wrt(last, sc0, sw0).wait()
        compute(last, qb0, kb0, sc0)
        wrt(last, sc0, sw0).start()
        gq(ri1, qb1, sq1).wait(); gk(ci1, kb1, sk1).wait()
        wrt(last, sc0, sw0).wait()
        wrt(last - 1, sc1, sw1).wait()

        mvec = jnp.full((16,), -1e30, _F32)
        zvec = jnp.zeros((16,), _F32)
        for h in range(_H):
            mv = m_buf[h, :]
            zv = z_buf[h, :]
            mt = jnp.max(mv)
            zt = jnp.sum(zv * jnp.exp(mv - jnp.full((16,), mt, _F32)))
            sel = iota == h
            mvec = jnp.where(sel, jnp.full((16,), mt, _F32), mvec)
            zvec = jnp.where(sel, jnp.full((16,), zt, _F32), zvec)
        mz_out[0, :] = mvec
        mz_out[1, :] = zvec
        pltpu.sync_copy(mz_out, mz_hbm.at[wid])

    return phase_a


def _phase_b_fn():
    out_type = jax.ShapeDtypeStruct((_NC, 2, _N, _HF), _F32)
    scratch = [
        pltpu.VMEM((_NCH, _B), _I32),
        pltpu.VMEM((_NCH, _B), _I32),
        pltpu.VMEM((_H, _B), _F32),
        pltpu.VMEM((_H, _B), _F32),
        pltpu.VMEM((_B, _HF), _F32),
        pltpu.VMEM((_B, _HF), _F32),
        pltpu.VMEM((_B, _HF), _F32),
        pltpu.VMEM((_B, _HF), _F32),
        pltpu.VMEM((_NW, 2, 16), _F32),
        pltpu.VMEM((_RC, _HF), _F32),
        pltpu.VMEM((_B,), _I32),
        pltpu.VMEM((_B,), _I32),
        pltpu.VMEM((_B,), _I32),
        pltpu.VMEM((_B,), _I32),
        pltpu.VMEM_SHARED((_N, _HF), _F32),
        pltpu.SemaphoreType.DMA,
        pltpu.SemaphoreType.DMA,
        pltpu.SemaphoreType.DMA,
        pltpu.SemaphoreType.DMA,
        pltpu.SemaphoreType.DMA,
        pltpu.SemaphoreType.DMA,
    ]

    @functools.partial(pl.kernel, out_type=out_type, mesh=_sc_mesh(),
                       scratch_types=scratch,
                       compiler_params=pltpu.CompilerParams(
                           needs_layout_passes=False, use_tc_tiling_on_sc=False))
    def phase_b(vlo_hbm, vhi_hbm, scores_hbm, mz_hbm, row_hbm, col_hbm,
                outp_hbm,
                row_all, col_all, sb0, sb1, vb0, vb1, mg0, mg1, mz_all,
                zrow_v, ri0, ri1, ci0, ci1, accum,
                sv0, sv1, sl0, sl1, ss0, ss1):
        cid = lax.axis_index("c")
        sid = lax.axis_index("s")
        wid = cid * _NS + sid

        pltpu.sync_copy(row_hbm.at[wid], row_all)
        pltpu.sync_copy(col_hbm.at[wid], col_all)

        pltpu.sync_copy(mz_hbm, mz_all)
        gm = mz_all[0, 0, :]
        for t in range(1, _NW):
            gm = jnp.maximum(gm, mz_all[t, 0, :])
        gz = jnp.zeros((16,), _F32)
        for t in range(_NW):
            gz = gz + mz_all[t, 1, :] * jnp.exp(mz_all[t, 0, :] - gm)
        ginv = 1.0 / gz
        gms = [jnp.full((16,), gm[h], _F32) for h in range(_H)]
        gis = [jnp.full((16,), ginv[h], _F32) for h in range(_H)]

        z16 = jnp.zeros((16,), _F32)

        def zr_body(r, _):
            for j in range(_HF // 16):
                zrow_v[r, pl.ds(j * 16, 16)] = z16
            return 0

        lax.fori_loop(0, _RC, zr_body, 0)

        def gsc(c, sb, sl):
            return pltpu.make_async_copy(scores_hbm.at[wid, c], sb, sl)

        def one_pass(half, v_hbm):
            heads = [half * _HH + j for j in range(_HH)]

            def zmsg_body(r, _):
                for j in range(_HF // 16):
                    mg0[r, pl.ds(j * 16, 16)] = z16
                    mg1[r, pl.ds(j * 16, 16)] = z16
                return 0

            lax.fori_loop(0, _B, zmsg_body, 0)
            for j in range(_RPT // _RC):
                pltpu.sync_copy(zrow_v,
                                accum.at[pl.ds(sid * _RPT + j * _RC, _RC)])
            plsc.subcore_barrier()

            def gv(ci, vb, sv):
                return pltpu.make_async_copy(v_hbm.at[ci], vb, sv)

            def sct(ri, mg, ss):
                return pltpu.make_async_copy(mg, accum.at[ri], ss)

            def issue_gv(c, ci, vb, sv):
                for g in range(_NG):
                    gs = g * 16
                    ci[pl.ds(gs, 16)] = col_all[c, pl.ds(gs, 16)]
                gv(ci, vb, sv).start()

            def issue_sct(c, ri, mg, ss):
                for g in range(_NG):
                    gs = g * 16
                    ri[pl.ds(gs, 16)] = row_all[c, pl.ds(gs, 16)]
                sct(ri, mg, ss).start(add=True)

            def compute(c, vb, sb, mg):
                def group_body(g, _):
                    gs = g * 16
                    attns = []
                    for h in heads:
                        s = sb[h, pl.ds(gs, 16)]
                        attns.append(jnp.exp(s - gms[h]) * gis[h])
                    for b in range(16):
                        e = gs + b
                        for j in range(_HH):
                            asc = jnp.full((16,), attns[j][b], _F32)
                            mg[e, pl.ds(j * _HD, _HD)] = (
                                vb[e, pl.ds(j * _HD, _HD)] * asc)
                    return 0

                lax.fori_loop(0, _NG, group_body, 0)

            issue_sct(0, ri0, mg0, ss0)
            issue_sct(1, ri1, mg1, ss1)
            issue_gv(0, ci0, vb0, sv0); gsc(0, sb0, sl0).start()
            issue_gv(1, ci1, vb1, sv1); gsc(1, sb1, sl1).start()

            def pair(j, _):
                c0 = 2 * j
                c1 = c0 + 1
                gv(ci0, vb0, sv0).wait(); gsc(c0, sb0, sl0).wait()
                sct(ri0, mg0, ss0).wait()
                compute(c0, vb0, sb0, mg0)
                issue_sct(c0, ri0, mg0, ss0)
                issue_gv(c0 + 2, ci0, vb0, sv0); gsc(c0 + 2, sb0, sl0).start()
                gv(ci1, vb1, sv1).wait(); gsc(c1, sb1, sl1).wait()
                sct(ri1, mg1, ss1).wait()
                compute(c1, vb1, sb1, mg1)
                issue_sct(c1, ri1, mg1, ss1)
                c3 = jnp.minimum(c1 + 2, _NCH - 1)
                issue_gv(c3, ci1, vb1, sv1); gsc(c3, sb1, sl1).start()
                return 0

            lax.fori_loop(0, (_NCH - 1) // 2, pair, 0)
            last = _NCH - 1
            gv(ci0, vb0, sv0).wait(); gsc(last, sb0, sl0).wait()
            sct(ri0, mg0, ss0).wait()
            compute(last, vb0, sb0, mg0)
            issue_sct(last, ri0, mg0, ss0)
            gv(ci1, vb1, sv1).wait(); gsc(last, sb1, sl1).wait()
            sct(ri0, mg0, ss0).wait()
            sct(ri1, mg1, ss1).wait()
            plsc.subcore_barrier()

            for j in range(_RPT // _RC):
                rb = sid * _RPT + j * _RC
                pltpu.sync_copy(accum.at[pl.ds(rb, _RC)],
                                outp_hbm.at[cid, half, pl.ds(rb, _RC)])

        one_pass(0, vlo_hbm)
        one_pass(1, vhi_hbm)

    return phase_b


_PHASE_A = _phase_a_fn()
_PHASE_B = _phase_b_fn()


@jax.jit
def _run(x, edge_index, edge_weight, Wq, bq, Wk, bk, Wv, bv, Wd, bd, Wo, bo):
    scale = 1.0 / math.sqrt(_HD)
    xp = jnp.pad(x, ((0, _NP - _N), (0, 0)))
    qs, ks, vs, xm = _dense(
        xp,
        Wq.T * scale, Wk.T, Wv.T,
        (bq * scale).reshape(1, _D), bk.reshape(1, _D), bv.reshape(1, _D),
    )
    row = edge_index[0].reshape(_NW, _NCH, _B)
    col = edge_index[1].reshape(_NW, _NCH, _B)
    ew3 = edge_weight.reshape(_NW, _NCH, _B)
    aux = (jnp.zeros((2, 16), _F32)
           .at[0, :_H].set(Wd[:, 0])
           .at[1, :_H].set(bd))
    scores, mz = _PHASE_A(qs, ks, xm, row, col, ew3, aux)
    partials = _PHASE_B(vs[:, :_HF], vs[:, _HF:], scores, mz, row, col)
    p = jnp.concatenate([partials[:, 0], partials[:, 1]], axis=-1)
    return _final(p[0], p[1], Wo.T, bo.reshape(1, _D))


def kernel(x, edge_index, edge_weight, Wq, bq, Wk, bk, Wv, bv, Wd, bd, Wo, bo):
    return _run(x, edge_index, edge_weight, Wq, bq, Wk, bk, Wv, bv,
                Wd, bd, Wo, bo)


# speedup vs baseline: 18.9623x; 1.0701x over previous
import functools
import math

import jax
import jax.numpy as jnp
from jax import lax
from jax.experimental import pallas as pl
from jax.experimental.pallas import tpu as pltpu
from jax.experimental.pallas import tpu_sc as plsc

_N = 10000
_NP = 10240
_E = 320000
_D = 128
_H = 8
_HD = 16
_NC = 2
_NS = 16
_NW = _NC * _NS
_EPT = _E // _NW
_B = 80
_NCH = _EPT // _B
_NG = _B // 16
_RPT = _N // _NS
_RC = 125
_HF = _D // 2
_DP = _D + 1
_HH = _H // 2

_F32 = jnp.float32
_I32 = jnp.int32


def _dense_body(x_ref, wq_ref, wk_ref, wv_ref, bq_ref, bk_ref, bv_ref,
                q_ref, k_ref, v_ref, xm_ref):
    xb = x_ref[...]
    q_ref[...] = (
        jnp.dot(xb, wq_ref[...], preferred_element_type=_F32) + bq_ref[...])
    k_ref[...] = (
        jnp.dot(xb, wk_ref[...], preferred_element_type=_F32) + bk_ref[...])
    v_ref[...] = jnp.dot(xb, wv_ref[...], preferred_element_type=_F32) + bv_ref[...]
    xm_ref[...] = jnp.mean(xb, axis=1).reshape(8, 128)


_BN = 1024


@jax.jit
def _dense(xp, wqt, wkt, wvt, bq2, bk2, bv2):
    grid = _NP // _BN
    return pl.pallas_call(
        _dense_body,
        grid=(grid,),
        in_specs=[
            pl.BlockSpec((_BN, _D), lambda i: (i, 0)),
            pl.BlockSpec((_D, _D), lambda i: (0, 0)),
            pl.BlockSpec((_D, _D), lambda i: (0, 0)),
            pl.BlockSpec((_D, _D), lambda i: (0, 0)),
            pl.BlockSpec((1, _D), lambda i: (0, 0)),
            pl.BlockSpec((1, _D), lambda i: (0, 0)),
            pl.BlockSpec((1, _D), lambda i: (0, 0)),
        ],
        out_specs=[
            pl.BlockSpec((_BN, _D), lambda i: (i, 0)),
            pl.BlockSpec((_BN, _D), lambda i: (i, 0)),
            pl.BlockSpec((_BN, _D), lambda i: (i, 0)),
            pl.BlockSpec((_BN // _D, _D), lambda i: (i, 0)),
        ],
        out_shape=[
            jax.ShapeDtypeStruct((_NP, _D), _F32),
            jax.ShapeDtypeStruct((_NP, _D), _F32),
            jax.ShapeDtypeStruct((_NP, _D), _F32),
            jax.ShapeDtypeStruct((_NP // _D, _D), _F32),
        ],
    )(xp, wqt, wkt, wvt, bq2, bk2, bv2)


def _final_body(p0_ref, p1_ref, wot_ref, bo_ref, o_ref):
    acc = p0_ref[...] + p1_ref[...]
    o_ref[...] = jnp.dot(acc, wot_ref[...], preferred_element_type=_F32) + bo_ref[...]


_FBN = 400


@jax.jit
def _final(p0, p1, wot, bo2):
    grid = _N // _FBN
    return pl.pallas_call(
        _final_body,
        grid=(grid,),
        in_specs=[
            pl.BlockSpec((_FBN, _D), lambda i: (i, 0)),
            pl.BlockSpec((_FBN, _D), lambda i: (i, 0)),
            pl.BlockSpec((_D, _D), lambda i: (0, 0)),
            pl.BlockSpec((1, _D), lambda i: (0, 0)),
        ],
        out_specs=pl.BlockSpec((_FBN, _D), lambda i: (i, 0)),
        out_shape=jax.ShapeDtypeStruct((_N, _D), _F32),
    )(p0, p1, wot, bo2)


def _sc_mesh():
    return plsc.VectorSubcoreMesh(
        core_axis_name="c", subcore_axis_name="s",
        num_cores=_NC, num_subcores=_NS)


def _phase_a_fn():
    out_type = (
        jax.ShapeDtypeStruct((_NW, _NCH, _H, _B), _F32),
        jax.ShapeDtypeStruct((_NW, 2, 16), _F32),
    )
    scratch = [
        pltpu.VMEM((_NCH, _B), _I32),
        pltpu.VMEM((_NCH, _B), _I32),
        pltpu.VMEM((_NCH, _B), _F32),
        pltpu.VMEM((_B, _D), _F32),
        pltpu.VMEM((_B, _D), _F32),
        pltpu.VMEM((_B, _D), _F32),
        pltpu.VMEM((_B, _D), _F32),
        pltpu.VMEM((_NP // _D, _D), _F32),
        pltpu.VMEM((2, 16), _F32),
        pltpu.VMEM((_H, _B), _F32),
        pltpu.VMEM((_H, _B), _F32),
        pltpu.VMEM((2, 16), _F32),
        pltpu.VMEM((_B,), _I32),
        pltpu.VMEM((_B,), _I32),
        pltpu.VMEM((_B,), _I32),
        pltpu.VMEM((_B,), _I32),
        pltpu.SemaphoreType.DMA,
        pltpu.SemaphoreType.DMA,
        pltpu.SemaphoreType.DMA,
        pltpu.SemaphoreType.DMA,
        pltpu.SemaphoreType.DMA,
        pltpu.SemaphoreType.DMA,
    ]

    @functools.partial(pl.kernel, out_type=out_type, mesh=_sc_mesh(),
                       scratch_types=scratch,
                       compiler_params=pltpu.CompilerParams(
                           needs_layout_passes=False, use_tc_tiling_on_sc=False))
    def phase_a(q_hbm, k_hbm, xm_hbm, row_hbm, col_hbm, ew_hbm, aux_hbm,
                scores_hbm, mz_hbm,
                row_all, col_all, ew_all, qb0, qb1, kb0, kb1, xm_v, aux_v,
                sc0, sc1, mz_out, ri0, ri1, ci0, ci1,
                sq0, sq1, sk0, sk1, sw0, sw1):
        cid = lax.axis_index("c")
        sid = lax.axis_index("s")
        wid = cid * _NS + sid

        pltpu.sync_copy(row_hbm.at[wid], row_all)
        pltpu.sync_copy(col_hbm.at[wid], col_all)
        pltpu.sync_copy(ew_hbm.at[wid], ew_all)
        pltpu.sync_copy(xm_hbm, xm_v)
        pltpu.sync_copy(aux_hbm, aux_v)

        neg = jnp.full((16,), -1e30, _F32)
        zero = jnp.zeros((16,), _F32)
        iota = lax.iota(_I32, 16)
        aux0 = aux_v[0, :]
        aux1 = aux_v[1, :]
        wds = [jnp.full((16,), aux0[h], _F32) for h in range(_H)]
        bds = [jnp.full((16,), aux1[h], _F32) for h in range(_H)]
        diags = [lax.bitwise_and(iota + d, 15) for d in range(_HD)]

        def gq(ri, qb, sq):
            return pltpu.make_async_copy(q_hbm.at[ri], qb, sq)

        def gk(ci, kb, sk):
            return pltpu.make_async_copy(k_hbm.at[ci], kb, sk)

        def stage_idx(c, ri, ci):
            for g in range(_NG):
                gs = g * 16
                ri[pl.ds(gs, 16)] = row_all[c, pl.ds(gs, 16)]
                ci[pl.ds(gs, 16)] = col_all[c, pl.ds(gs, 16)]

        def issue(c, ri, ci, qb, kb, sq, sk):
            stage_idx(c, ri, ci)
            gq(ri, qb, sq).start()
            gk(ci, kb, sk).start()

        def wrt(c, scb, sw):
            return pltpu.make_async_copy(scb, scores_hbm.at[wid, c], sw)

        def compute(c, qb, kb, scb, mz):
            def group_body(g, mz_c):
                ms, zs = mz_c
                gs = g * 16
                row16 = row_all[c, pl.ds(gs, 16)]
                col16 = col_all[c, pl.ds(gs, 16)]
                ew16 = ew_all[c, pl.ds(gs, 16)]
                dsc = (plsc.load_gather(
                           xm_v, [lax.shift_right_logical(col16, 7),
                                  lax.bitwise_and(col16, 127)])
                       - plsc.load_gather(
                           xm_v, [lax.shift_right_logical(row16, 7),
                                  lax.bitwise_and(row16, 127)]))
                lidx = iota + gs
                ms2 = []
                zs2 = []
                for h in range(_H):
                    acc = dsc * wds[h] + ew16 + bds[h]
                    for d in range(_HD):
                        cix = diags[d] + (h * _HD)
                        acc = acc + (plsc.load_gather(qb, [lidx, cix])
                                     * plsc.load_gather(kb, [lidx, cix]))
                    scb[h, pl.ds(gs, 16)] = acc
                    m_new = jnp.maximum(ms[h], acc)
                    ms2.append(m_new)
                    zs2.append(zs[h] * jnp.exp(ms[h] - m_new)
                               + jnp.exp(acc - m_new))
                return tuple(ms2), tuple(zs2)

            return lax.fori_loop(0, _NG, group_body, mz)

        issue(0, ri0, ci0, qb0, kb0, sq0, sk0)
        issue(1, ri1, ci1, qb1, kb1, sq1, sk1)
        wrt(0, sc0, sw0).start()
        wrt(1, sc1, sw1).start()

        def pair(j, mz):
            c0 = 2 * j
            c1 = c0 + 1
            gq(ri0, qb0, sq0).wait(); gk(ci0, kb0, sk0).wait()
            wrt(c0, sc0, sw0).wait()
            mz = compute(c0, qb0, kb0, sc0, mz)
            wrt(c0, sc0, sw0).start()
            issue(c0 + 2, ri0, ci0, qb0, kb0, sq0, sk0)
            gq(ri1, qb1, sq1).wait(); gk(ci1, kb1, sk1).wait()
            wrt(c1, sc1, sw1).wait()
            mz = compute(c1, qb1, kb1, sc1, mz)
            wrt(c1, sc1, sw1).start()
            c3 = jnp.minimum(c1 + 2, _NCH - 1)
            issue(c3, ri1, ci1, qb1, kb1, sq1, sk1)
            return mz

        mz0 = (tuple(neg for _ in range(_H)), tuple(zero for _ in range(_H)))
        mz = lax.fori_loop(0, (_NCH - 1) // 2, pair, mz0)
        last = _NCH - 1
        gq(ri0, qb0, sq0).wait(); gk(ci0, kb0, sk0).wait()
        wrt(last, sc0, sw0).wait()
        mz = compute(last, qb0, kb0, sc0, mz)
        wrt(last, sc0, sw0).start()
        gq(ri1, qb1, sq1).wait(); gk(ci1, kb1, sk1).wait()
        wrt(last, sc0, sw0).wait()
        wrt(last - 1, sc1, sw1).wait()
        ms_f, zs_f = mz

        mvec = jnp.full((16,), -1e30, _F32)
        zvec = jnp.zeros((16,), _F32)
        for h in range(_H):
            mv = ms_f[h]
            zv = zs_f[h]
            mt = jnp.max(mv)
            zt = jnp.sum(zv * jnp.exp(mv - jnp.full((16,), mt, _F32)))
            sel = iota == h
            mvec = jnp.where(sel, jnp.full((16,), mt, _F32), mvec)
            zvec = jnp.where(sel, jnp.full((16,), zt, _F32), zvec)
        mz_out[0, :] = mvec
        mz_out[1, :] = zvec
        pltpu.sync_copy(mz_out, mz_hbm.at[wid])

    return phase_a


def _phase_b_fn():
    out_type = jax.ShapeDtypeStruct((_NC, 2, _N, _HF), _F32)
    scratch = [
        pltpu.VMEM((_NCH, _B), _I32),
        pltpu.VMEM((_NCH, _B), _I32),
        pltpu.VMEM((_H, _B), _F32),
        pltpu.VMEM((_H, _B), _F32),
        pltpu.VMEM((_B, _HF), _F32),
        pltpu.VMEM((_B, _HF), _F32),
        pltpu.VMEM((_B, _HF), _F32),
        pltpu.VMEM((_B, _HF), _F32),
        pltpu.VMEM((_NW, 2, 16), _F32),
        pltpu.VMEM((_RC, _HF), _F32),
        pltpu.VMEM((_B,), _I32),
        pltpu.VMEM((_B,), _I32),
        pltpu.VMEM((_B,), _I32),
        pltpu.VMEM((_B,), _I32),
        pltpu.VMEM_SHARED((_N, _HF), _F32),
        pltpu.SemaphoreType.DMA,
        pltpu.SemaphoreType.DMA,
        pltpu.SemaphoreType.DMA,
        pltpu.SemaphoreType.DMA,
        pltpu.SemaphoreType.DMA,
        pltpu.SemaphoreType.DMA,
    ]

    @functools.partial(pl.kernel, out_type=out_type, mesh=_sc_mesh(),
                       scratch_types=scratch,
                       compiler_params=pltpu.CompilerParams(
                           needs_layout_passes=False, use_tc_tiling_on_sc=False))
    def phase_b(vlo_hbm, vhi_hbm, scores_hbm, mz_hbm, row_hbm, col_hbm,
                outp_hbm,
                row_all, col_all, sb0, sb1, vb0, vb1, mg0, mg1, mz_all,
                zrow_v, ri0, ri1, ci0, ci1, accum,
                sv0, sv1, sl0, sl1, ss0, ss1):
        cid = lax.axis_index("c")
        sid = lax.axis_index("s")
        wid = cid * _NS + sid

        pltpu.sync_copy(row_hbm.at[wid], row_all)
        pltpu.sync_copy(col_hbm.at[wid], col_all)

        pltpu.sync_copy(mz_hbm, mz_all)
        gm = mz_all[0, 0, :]
        for t in range(1, _NW):
            gm = jnp.maximum(gm, mz_all[t, 0, :])
        gz = jnp.zeros((16,), _F32)
        for t in range(_NW):
            gz = gz + mz_all[t, 1, :] * jnp.exp(mz_all[t, 0, :] - gm)
        ginv = 1.0 / gz
        gms = [jnp.full((16,), gm[h], _F32) for h in range(_H)]
        gis = [jnp.full((16,), ginv[h], _F32) for h in range(_H)]

        z16 = jnp.zeros((16,), _F32)

        def zr_body(r, _):
            for j in range(_HF // 16):
                zrow_v[r, pl.ds(j * 16, 16)] = z16
            return 0

        lax.fori_loop(0, _RC, zr_body, 0)

        def gsc(c, sb, sl):
            return pltpu.make_async_copy(scores_hbm.at[wid, c], sb, sl)

        def one_pass(half, v_hbm):
            heads = [half * _HH + j for j in range(_HH)]

            def zmsg_body(r, _):
                for j in range(_HF // 16):
                    mg0[r, pl.ds(j * 16, 16)] = z16
                    mg1[r, pl.ds(j * 16, 16)] = z16
                return 0

            lax.fori_loop(0, _B, zmsg_body, 0)
            for j in range(_RPT // _RC):
                pltpu.sync_copy(zrow_v,
                                accum.at[pl.ds(sid * _RPT + j * _RC, _RC)])
            plsc.subcore_barrier()

            def gv(ci, vb, sv):
                return pltpu.make_async_copy(v_hbm.at[ci], vb, sv)

            def sct(ri, mg, ss):
                return pltpu.make_async_copy(mg, accum.at[ri], ss)

            def issue_gv(c, ci, vb, sv):
                for g in range(_NG):
                    gs = g * 16
                    ci[pl.ds(gs, 16)] = col_all[c, pl.ds(gs, 16)]
                gv(ci, vb, sv).start()

            def issue_sct(c, ri, mg, ss):
                for g in range(_NG):
                    gs = g * 16
                    ri[pl.ds(gs, 16)] = row_all[c, pl.ds(gs, 16)]
                sct(ri, mg, ss).start(add=True)

            def compute(c, vb, sb, mg):
                def group_body(g, _):
                    gs = g * 16
                    attns = []
                    for h in heads:
                        s = sb[h, pl.ds(gs, 16)]
                        attns.append(jnp.exp(s - gms[h]) * gis[h])
                    for b in range(16):
                        e = gs + b
                        for j in range(_HH):
                            asc = jnp.full((16,), attns[j][b], _F32)
                            mg[e, pl.ds(j * _HD, _HD)] = (
                                vb[e, pl.ds(j * _HD, _HD)] * asc)
                    return 0

                lax.fori_loop(0, _NG, group_body, 0)

            issue_sct(0, ri0, mg0, ss0)
            issue_sct(1, ri1, mg1, ss1)
            issue_gv(0, ci0, vb0, sv0); gsc(0, sb0, sl0).start()
            issue_gv(1, ci1, vb1, sv1); gsc(1, sb1, sl1).start()

            def pair(j, _):
                c0 = 2 * j
                c1 = c0 + 1
                gv(ci0, vb0, sv0).wait(); gsc(c0, sb0, sl0).wait()
                sct(ri0, mg0, ss0).wait()
                compute(c0, vb0, sb0, mg0)
                issue_sct(c0, ri0, mg0, ss0)
                issue_gv(c0 + 2, ci0, vb0, sv0); gsc(c0 + 2, sb0, sl0).start()
                gv(ci1, vb1, sv1).wait(); gsc(c1, sb1, sl1).wait()
                sct(ri1, mg1, ss1).wait()
                compute(c1, vb1, sb1, mg1)
                issue_sct(c1, ri1, mg1, ss1)
                c3 = jnp.minimum(c1 + 2, _NCH - 1)
                issue_gv(c3, ci1, vb1, sv1); gsc(c3, sb1, sl1).start()
                return 0

            lax.fori_loop(0, (_NCH - 1) // 2, pair, 0)
            last = _NCH - 1
            gv(ci0, vb0, sv0).wait(); gsc(last, sb0, sl0).wait()
            sct(ri0, mg0, ss0).wait()
            compute(last, vb0, sb0, mg0)
            issue_sct(last, ri0, mg0, ss0)
            gv(ci1, vb1, sv1).wait(); gsc(last, sb1, sl1).wait()
            sct(ri0, mg0, ss0).wait()
            sct(ri1, mg1, ss1).wait()
            plsc.subcore_barrier()

            for j in range(_RPT // _RC):
                rb = sid * _RPT + j * _RC
                pltpu.sync_copy(accum.at[pl.ds(rb, _RC)],
                                outp_hbm.at[cid, half, pl.ds(rb, _RC)])

        one_pass(0, vlo_hbm)
        one_pass(1, vhi_hbm)

    return phase_b


_PHASE_A = _phase_a_fn()
_PHASE_B = _phase_b_fn()


@jax.jit
def _run(x, edge_index, edge_weight, Wq, bq, Wk, bk, Wv, bv, Wd, bd, Wo, bo):
    scale = 1.0 / math.sqrt(_HD)
    xp = jnp.pad(x, ((0, _NP - _N), (0, 0)))
    qs, ks, vs, xm = _dense(
        xp,
        Wq.T * scale, Wk.T, Wv.T,
        (bq * scale).reshape(1, _D), bk.reshape(1, _D), bv.reshape(1, _D),
    )
    row = edge_index[0].reshape(_NW, _NCH, _B)
    col = edge_index[1].reshape(_NW, _NCH, _B)
    ew3 = edge_weight.reshape(_NW, _NCH, _B)
    aux = (jnp.zeros((2, 16), _F32)
           .at[0, :_H].set(Wd[:, 0])
           .at[1, :_H].set(bd))
    scores, mz = _PHASE_A(qs, ks, xm, row, col, ew3, aux)
    partials = _PHASE_B(vs[:, :_HF], vs[:, _HF:], scores, mz, row, col)
    p = jnp.concatenate([partials[:, 0], partials[:, 1]], axis=-1)
    return _final(p[0], p[1], Wo.T, bo.reshape(1, _D))


def kernel(x, edge_index, edge_weight, Wq, bq, Wk, bk, Wv, bv, Wd, bd, Wo, bo):
    return _run(x, edge_index, edge_weight, Wq, bq, Wk, bk, Wv, bv,
                Wd, bd, Wo, bo)
